# baseline (speedup 1.0000x reference)
import functools

import jax
import jax.numpy as jnp
from jax import lax
from jax.experimental import pallas as pl
from jax.experimental.pallas import tpu as pltpu
from jax.experimental.pallas import tpu_sc as plsc

_RB = 4096


def _vq_assign_body(z_ref, cb_ref, idx_ref, loss_ref):
    zb = z_ref[...]
    cb = cb_ref[...]
    k, dd = cb.shape
    s2t = lax.dot_general(
        cb + cb, zb, (((1,), (1,)), ((), ())),
        preferred_element_type=jnp.float32)
    zt = zb.T
    t = zt * zt
    half = dd
    while half > 1:
        half //= 2
        t = t[:half] + t[half:2 * half]
    zsq_row = t
    csq_col = jnp.sum(cb * cb, axis=1, keepdims=True)
    d = (zsq_row - s2t) + csq_col
    dmin_row = jnp.min(d, axis=0, keepdims=True)
    code = lax.broadcasted_iota(jnp.int32, d.shape, 0)
    cand = jnp.where(d == dmin_row, code, k)
    idx_ref[...] = jnp.min(cand, axis=0)

    @pl.when(pl.program_id(0) == 0)
    def _():
        loss_ref[0, 0] = 0.0

    loss_ref[0, 0] += jnp.sum(dmin_row)


def _vq_assign(flat, codebook):
    n, d = flat.shape
    k = codebook.shape[0]
    return pl.pallas_call(
        _vq_assign_body,
        grid=(n // _RB,),
        in_specs=[
            pl.BlockSpec((_RB, d), lambda i: (i, 0)),
            pl.BlockSpec((k, d), lambda i: (0, 0)),
        ],
        out_specs=[
            pl.BlockSpec((_RB,), lambda i: (i,)),
            pl.BlockSpec((1, 1), lambda i: (0, 0), memory_space=pltpu.SMEM),
        ],
        out_shape=[
            jax.ShapeDtypeStruct((n,), jnp.int32),
            jax.ShapeDtypeStruct((1, 1), jnp.float32),
        ],
    )(flat, codebook)


def _sc_gather(codebook, idx, b, s, d):
    n = idx.shape[0]
    k = codebook.shape[0]
    info = plsc.get_sparse_core_info()
    nc, ns, nl = info.num_cores, info.num_subcores, info.num_lanes
    nw = nc * ns
    bpw = n // nw
    ch = s
    nchunk = bpw // ch
    groups = ch // nl
    mesh = plsc.VectorSubcoreMesh(core_axis_name="c", subcore_axis_name="s")

    @functools.partial(
        pl.kernel,
        mesh=mesh,
        compiler_params=pltpu.CompilerParams(
            use_tc_tiling_on_sc=False, needs_layout_passes=False),
        out_type=jax.ShapeDtypeStruct((b, s * d), jnp.float32),
        scratch_types=[
            pltpu.VMEM((k * d,), jnp.float32),
            pltpu.VMEM((bpw,), jnp.int32),
            pltpu.VMEM((ch * d,), jnp.float32),
        ],
    )
    def gather(cb_hbm, idx_hbm, out_hbm, cb_v, idx_v, out_v):
        wid = lax.axis_index("s") * nc + lax.axis_index("c")
        pltpu.sync_copy(cb_hbm, cb_v)
        pltpu.sync_copy(idx_hbm.at[pl.ds(wid * bpw, bpw)], idx_v)
        lane = jnp.arange(nl, dtype=jnp.int32)

        b0 = wid * nchunk
        for j in range(nchunk):

            def group_body(g, carry):
                kvec = idx_v[pl.ds((j * groups + g) * nl, nl)]
                kbase = kvec * d
                obase = (g * nl + lane) * d
                for c in range(d):
                    val = plsc.load_gather(cb_v, [kbase + c])
                    plsc.store_scatter(out_v, [obase + c], val)
                return carry

            lax.fori_loop(0, groups, group_body, 0, unroll=False)
            pltpu.sync_copy(out_v, out_hbm.at[b0 + j])

    return gather(codebook.reshape(k * d), idx)


def kernel(z, codebook):
    b, s, d = z.shape
    n = b * s
    flat = z.reshape(n, d)
    idx, loss = _vq_assign(flat, codebook)
    quantized = _sc_gather(codebook, idx, b, s, d).reshape(b, s, d)
    mse = loss[0, 0] / jnp.float32(n * d)
    return quantized, 0.25 * mse, mse


# speedup vs baseline: 1.3838x; 1.3838x over previous
import functools

import jax
import jax.numpy as jnp
from jax import lax
from jax.experimental import pallas as pl
from jax.experimental.pallas import tpu as pltpu
from jax.experimental.pallas import tpu_sc as plsc

_RB = 4096


def _vq_assign_body(z_ref, cb_ref, idx_ref, loss_ref):
    zb = z_ref[...]
    cb = cb_ref[...]
    k, dd = cb.shape
    s2t = lax.dot_general(
        cb + cb, zb, (((1,), (1,)), ((), ())),
        preferred_element_type=jnp.float32)
    zt = zb.T
    t = zt * zt
    half = dd
    while half > 1:
        half //= 2
        t = t[:half] + t[half:2 * half]
    zsq_row = t
    csq_col = jnp.sum(cb * cb, axis=1, keepdims=True)
    d = (zsq_row - s2t) + csq_col
    dmin_row = jnp.min(d, axis=0, keepdims=True)
    code = lax.broadcasted_iota(jnp.int32, d.shape, 0).astype(jnp.float32)
    cand = jnp.where(d == dmin_row, code, jnp.float32(k))
    idx_ref[...] = jnp.min(cand, axis=0).astype(jnp.int32)

    @pl.when(pl.program_id(0) == 0)
    def _():
        loss_ref[0, 0] = 0.0

    loss_ref[0, 0] += jnp.sum(dmin_row)


def _vq_assign(flat, codebook):
    n, d = flat.shape
    k = codebook.shape[0]
    return pl.pallas_call(
        _vq_assign_body,
        grid=(n // _RB,),
        in_specs=[
            pl.BlockSpec((_RB, d), lambda i: (i, 0)),
            pl.BlockSpec((k, d), lambda i: (0, 0)),
        ],
        out_specs=[
            pl.BlockSpec((_RB,), lambda i: (i,)),
            pl.BlockSpec((1, 1), lambda i: (0, 0), memory_space=pltpu.SMEM),
        ],
        out_shape=[
            jax.ShapeDtypeStruct((n,), jnp.int32),
            jax.ShapeDtypeStruct((1, 1), jnp.float32),
        ],
    )(flat, codebook)


def _sc_gather(codebook, idx, b, s, d):
    n = idx.shape[0]
    info = plsc.get_sparse_core_info()
    nc, ns = info.num_cores, info.num_subcores
    nw = nc * ns
    bpw = n // nw
    ch = s
    nchunk = bpw // ch
    mesh = plsc.VectorSubcoreMesh(core_axis_name="c", subcore_axis_name="s")

    @functools.partial(
        pl.kernel,
        mesh=mesh,
        compiler_params=pltpu.CompilerParams(use_tc_tiling_on_sc=False),
        out_type=jax.ShapeDtypeStruct((b, s, d), jnp.float32),
        scratch_types=[
            pltpu.VMEM((bpw,), jnp.int32),
            pltpu.VMEM((ch, d), jnp.float32),
            pltpu.VMEM((ch, d), jnp.float32),
            pltpu.SemaphoreType.DMA,
            pltpu.SemaphoreType.DMA,
        ],
    )
    def gather(cb_hbm, idx_hbm, out_hbm, idx_v, rows0, rows1, sem0, sem1):
        wid = lax.axis_index("s") * nc + lax.axis_index("c")
        b0 = wid * nchunk
        pltpu.sync_copy(idx_hbm.at[pl.ds(wid * bpw, bpw)], idx_v)
        bufs = (rows0, rows1)
        sems = (sem0, sem1)
        prev = pltpu.async_copy(
            cb_hbm.at[idx_v.at[pl.ds(0, ch)]], bufs[0], sems[0])
        for j in range(nchunk):
            if j + 1 < nchunk:
                nxt = pltpu.async_copy(
                    cb_hbm.at[idx_v.at[pl.ds((j + 1) * ch, ch)]],
                    bufs[(j + 1) % 2], sems[(j + 1) % 2])
            prev.wait()
            pltpu.sync_copy(bufs[j % 2], out_hbm.at[b0 + j])
            if j + 1 < nchunk:
                prev = nxt

    return gather(codebook, idx)


def kernel(z, codebook):
    b, s, d = z.shape
    n = b * s
    flat = z.reshape(n, d)
    idx, loss = _vq_assign(flat, codebook)
    quantized = _sc_gather(codebook, idx, b, s, d)
    mse = loss[0, 0] / jnp.float32(n * d)
    return quantized, 0.25 * mse, mse
